# baseline (speedup 1.0000x reference)
import functools

import jax
import jax.numpy as jnp
from jax import lax
from jax.experimental import pallas as pl
from jax.experimental.pallas import tpu as pltpu
from jax.experimental.pallas import tpu_sc as plsc

N = 10000
E = 320000
D = 128

NC, NS = 2, 16
NW = NC * NS
EPW = E // NW
K = 40
CH = EPW // K
RPS = 640
RB = 2000
GRID = N // RB


S = 5
NRND = CH // S


def _make_sc_aggregate(with_count):
  mesh = plsc.VectorSubcoreMesh(core_axis_name="c", subcore_axis_name="s")
  out_type = [jax.ShapeDtypeStruct((NC, N, D), jnp.float32)]
  scratch = (
      [pltpu.VMEM((EPW,), jnp.int32)] +
      [pltpu.VMEM((K,), jnp.int32)] * (2 * S) +
      [pltpu.VMEM((K, D), jnp.float32)] * S +
      [pltpu.SemaphoreType.DMA] * (4 * S + 1) +
      [pltpu.VMEM_SHARED((N, D), jnp.float32)]
  )
  if with_count:
    out_type.append(jax.ShapeDtypeStruct((N,), jnp.float32))
    out_type.append(jax.ShapeDtypeStruct((N,), jnp.float32))
    scratch += [
        pltpu.VMEM((48,), jnp.float32),
        pltpu.VMEM((RPS,), jnp.float32),
        pltpu.VMEM_SHARED((N,), jnp.float32),
    ]

  def body(x_hbm, src_hbm, dst_hbm, *rest):
    if with_count:
      sum_hbm, cnt0_hbm, cnt1_hbm = rest[:3]
      rest = rest[3:]
    else:
      (sum_hbm,) = rest[:1]
      rest = rest[1:]
    srcv = rest[0]
    idxd = (rest[1:1 + S], rest[1 + S:1 + 2 * S])
    rows = rest[1 + 2 * S:1 + 3 * S]
    rest = rest[1 + 3 * S:]
    isem = (rest[:S], rest[S:2 * S])
    gsem = rest[2 * S:3 * S]
    ssem = rest[3 * S:4 * S]
    zsem = rest[4 * S]
    acc_sh = rest[4 * S + 1]
    if with_count:
      ones, cntv, cnt_sh = rest[4 * S + 2:]
    cid = lax.axis_index("c")
    sid = lax.axis_index("s")
    wid = sid * NC + cid
    base = jnp.minimum(sid * RPS, N - RPS)
    for r in range(K):
      for j in range(D // 16):
        rows[0][r, pl.ds(j * 16, 16)] = jnp.zeros((16,), jnp.float32)
    for t in range(RPS // K):
      pltpu.async_copy(rows[0], acc_sh.at[pl.ds(base + t * K, K)], zsem)
    if with_count:
      for i in range(RPS // 16):
        cntv[pl.ds(i * 16, 16)] = jnp.zeros((16,), jnp.float32)
      pltpu.sync_copy(cntv, cnt_sh.at[pl.ds(base, RPS)])
      for i in range(3):
        ones[pl.ds(i * 16, 16)] = jnp.ones((16,), jnp.float32)
    for t in range(RPS // K):
      pltpu.make_async_copy(rows[0], acc_sh.at[pl.ds(base, K)], zsem).wait()
    plsc.subcore_barrier()

    ebase = wid * EPW
    pltpu.sync_copy(src_hbm.at[pl.ds(ebase, EPW)], srcv)

    def i_start(c, f, p):
      pltpu.async_copy(dst_hbm.at[pl.ds(ebase + c * K, K)],
                       idxd[f][p], isem[f][p])

    def i_wait(f, p):
      pltpu.make_async_copy(dst_hbm.at[pl.ds(ebase, K)],
                            idxd[f][p], isem[f][p]).wait()

    def g_start(c, p):
      pltpu.async_copy(x_hbm.at[srcv.at[pl.ds(c * K, K)]], rows[p], gsem[p])

    def g_wait(p):
      pltpu.make_async_copy(x_hbm.at[srcv.at[pl.ds(0, K)]], rows[p],
                            gsem[p]).wait()

    def s_start(f, p):
      pltpu.async_copy(rows[p], acc_sh.at[idxd[f][p]], ssem[p], add=True)

    def s_wait(f, p):
      pltpu.make_async_copy(rows[p], acc_sh.at[idxd[f][p]], ssem[p]).wait()

    def c_start(f, p):
      pltpu.async_copy(ones.at[pl.ds(0, K)], cnt_sh.at[idxd[f][p]],
                       ssem[p], add=True)

    def c_wait(f, p):
      pltpu.make_async_copy(ones.at[pl.ds(0, K)], cnt_sh.at[idxd[f][p]],
                            ssem[p]).wait()

    for p in range(S):
      i_start(p, 0, p)

    def loop_body(j, carry):
      for f in range(2):
        cbase = (2 * j + f) * S
        for p in range(S):
          if f == 0:
            @pl.when(j > 0)
            def _():
              s_wait(f, p)
              if with_count:
                c_wait(f, p)
          else:
            s_wait(f, p)
            if with_count:
              c_wait(f, p)
          i_start(jnp.minimum(cbase + S + p, CH - 1), 1 - f, p)
          i_wait(f, p)
          g_start(cbase + p, p)
        for p in range(S):
          g_wait(p)
          s_start(f, p)
          if with_count:
            c_start(f, p)
      return carry

    lax.fori_loop(0, NRND // 2, loop_body, 0)
    for p in range(S):
      s_wait(1, p)
      if with_count:
        c_wait(1, p)
      i_wait(0, p)
    plsc.subcore_barrier()
    pltpu.sync_copy(acc_sh.at[pl.ds(base, RPS)],
                    sum_hbm.at[cid, pl.ds(base, RPS)])
    if with_count:
      pltpu.sync_copy(cnt_sh.at[pl.ds(base, RPS)], cntv)
      @pl.when(cid == 0)
      def _():
        pltpu.sync_copy(cntv, cnt0_hbm.at[pl.ds(base, RPS)])
      @pl.when(cid == 1)
      def _():
        pltpu.sync_copy(cntv, cnt1_hbm.at[pl.ds(base, RPS)])

  return pl.kernel(body, out_type=out_type, mesh=mesh, scratch_types=scratch)


_sc_agg_count = _make_sc_aggregate(True)
_sc_agg = _make_sc_aggregate(False)


def _tc_split_body(ei_ref, src_ref, dst_ref):
  src_ref[...] = ei_ref[0]
  dst_ref[...] = ei_ref[1]


_tc_split = pl.pallas_call(
    _tc_split_body,
    out_shape=[jax.ShapeDtypeStruct((E,), jnp.int32)] * 2,
)


def _tc_layer_body(has_final, sp_ref, cnt0_ref, cnt1_ref, x_ref, Wl_ref,
                   bl_ref, Wr_ref, *rest):
  if has_final:
    Wf_ref, bf_ref, out_ref = rest
  else:
    (out_ref,) = rest
  s = sp_ref[0] + sp_ref[1]
  c = jnp.maximum(cnt0_ref[0, 0] + cnt1_ref[0, 0], 1.0)
  agg = s / c[:, None]
  dn = (((1,), (1,)), ((), ()))
  h = lax.dot_general(agg, Wl_ref[...], dn, preferred_element_type=jnp.float32)
  h = h + bl_ref[...] + lax.dot_general(
      x_ref[...], Wr_ref[...], dn, preferred_element_type=jnp.float32)
  h = jnp.maximum(h, 0.0)
  if has_final:
    h = lax.dot_general(h, Wf_ref[...], dn,
                        preferred_element_type=jnp.float32) + bf_ref[...]
  out_ref[...] = h


def _make_tc_layer(has_final):
  wspec = pl.BlockSpec((D, D), lambda i: (0, 0))
  bspec = pl.BlockSpec((1, D), lambda i: (0, 0))
  cspec = pl.BlockSpec((1, 1, RB), lambda i: (i, 0, 0))
  in_specs = [
      pl.BlockSpec((NC, RB, D), lambda i: (0, i, 0)),
      cspec, cspec,
      pl.BlockSpec((RB, D), lambda i: (i, 0)),
      wspec, bspec, wspec,
  ]
  if has_final:
    in_specs += [wspec, bspec]
  return pl.pallas_call(
      functools.partial(_tc_layer_body, has_final),
      grid=(GRID,),
      in_specs=in_specs,
      out_specs=pl.BlockSpec((RB, D), lambda i: (i, 0)),
      out_shape=jax.ShapeDtypeStruct((N, D), jnp.float32),
  )


_tc_layer = _make_tc_layer(False)
_tc_layer_final = _make_tc_layer(True)


def kernel(x, edge_index, W1l, b1l, W1r, W2l, b2l, W2r, Wf, bf):
  src, dst = _tc_split(edge_index)
  sum1, cnt0, cnt1 = _sc_agg_count(x, src, dst)
  cnt0 = cnt0.reshape(GRID, 1, RB)
  cnt1 = cnt1.reshape(GRID, 1, RB)
  h1 = _tc_layer(sum1, cnt0, cnt1, x, W1l, b1l.reshape(1, D), W1r)
  (sum2,) = _sc_agg(h1, src, dst)
  return _tc_layer_final(sum2, cnt0, cnt1, h1, W2l, b2l.reshape(1, D), W2r,
                         Wf, bf.reshape(1, D))


# speedup vs baseline: 1.0085x; 1.0085x over previous
import functools

import jax
import jax.numpy as jnp
from jax import lax
from jax.experimental import pallas as pl
from jax.experimental.pallas import tpu as pltpu
from jax.experimental.pallas import tpu_sc as plsc

N = 10000
E = 320000
D = 128

NC, NS = 2, 16
NW = NC * NS
EPW = E // NW
K = 40
CH = EPW // K
RPS = 640
RB = 2000
GRID = N // RB


S = 5
NRND = CH // S


def _make_sc_aggregate(with_count):
  mesh = plsc.VectorSubcoreMesh(core_axis_name="c", subcore_axis_name="s")
  out_type = [jax.ShapeDtypeStruct((NC, N, D), jnp.float32)]
  scratch = (
      [pltpu.VMEM((EPW,), jnp.int32)] +
      [pltpu.VMEM((K,), jnp.int32)] * (2 * S) +
      [pltpu.VMEM((K, D), jnp.float32)] * S +
      [pltpu.SemaphoreType.DMA] * (4 * S + 2) +
      [pltpu.VMEM_SHARED((N, D), jnp.float32)]
  )
  if with_count:
    out_type.append(jax.ShapeDtypeStruct((N,), jnp.float32))
    out_type.append(jax.ShapeDtypeStruct((N,), jnp.float32))
    scratch += [
        pltpu.VMEM((48,), jnp.float32),
        pltpu.VMEM((RPS,), jnp.float32),
        pltpu.VMEM_SHARED((N,), jnp.float32),
    ]

  def body(x_hbm, src_hbm, dst_hbm, *rest):
    if with_count:
      sum_hbm, cnt0_hbm, cnt1_hbm = rest[:3]
      rest = rest[3:]
    else:
      (sum_hbm,) = rest[:1]
      rest = rest[1:]
    srcv = rest[0]
    idxd = (rest[1:1 + S], rest[1 + S:1 + 2 * S])
    rows = rest[1 + 2 * S:1 + 3 * S]
    rest = rest[1 + 3 * S:]
    isem = (rest[:S], rest[S:2 * S])
    gsem = rest[2 * S:3 * S]
    ssem = rest[3 * S:4 * S]
    zsem = rest[4 * S]
    xsem = rest[4 * S + 1]
    acc_sh = rest[4 * S + 2]
    if with_count:
      ones, cntv, cnt_sh = rest[4 * S + 3:]
    cid = lax.axis_index("c")
    sid = lax.axis_index("s")
    wid = sid * NC + cid
    base = jnp.minimum(sid * RPS, N - RPS)
    pltpu.async_copy(src_hbm.at[pl.ds(wid * EPW, EPW)], srcv, xsem)
    for r in range(K):
      for j in range(D // 16):
        rows[0][r, pl.ds(j * 16, 16)] = jnp.zeros((16,), jnp.float32)
    for t in range(RPS // K):
      pltpu.async_copy(rows[0], acc_sh.at[pl.ds(base + t * K, K)], zsem)
    if with_count:
      for i in range(RPS // 16):
        cntv[pl.ds(i * 16, 16)] = jnp.zeros((16,), jnp.float32)
      pltpu.sync_copy(cntv, cnt_sh.at[pl.ds(base, RPS)])
      for i in range(3):
        ones[pl.ds(i * 16, 16)] = jnp.ones((16,), jnp.float32)
    for t in range(RPS // K):
      pltpu.make_async_copy(rows[0], acc_sh.at[pl.ds(base, K)], zsem).wait()
    ebase = wid * EPW
    pltpu.make_async_copy(src_hbm.at[pl.ds(ebase, EPW)], srcv, xsem).wait()
    plsc.subcore_barrier()

    def i_start(c, f, p):
      pltpu.async_copy(dst_hbm.at[pl.ds(ebase + c * K, K)],
                       idxd[f][p], isem[f][p])

    def i_wait(f, p):
      pltpu.make_async_copy(dst_hbm.at[pl.ds(ebase, K)],
                            idxd[f][p], isem[f][p]).wait()

    def g_start(c, p):
      pltpu.async_copy(x_hbm.at[srcv.at[pl.ds(c * K, K)]], rows[p], gsem[p])

    def g_wait(p):
      pltpu.make_async_copy(x_hbm.at[srcv.at[pl.ds(0, K)]], rows[p],
                            gsem[p]).wait()

    def s_start(f, p):
      pltpu.async_copy(rows[p], acc_sh.at[idxd[f][p]], ssem[p], add=True)

    def s_wait(f, p):
      pltpu.make_async_copy(rows[p], acc_sh.at[idxd[f][p]], ssem[p]).wait()

    def c_start(f, p):
      pltpu.async_copy(ones.at[pl.ds(0, K)], cnt_sh.at[idxd[f][p]],
                       ssem[p], add=True)

    def c_wait(f, p):
      pltpu.make_async_copy(ones.at[pl.ds(0, K)], cnt_sh.at[idxd[f][p]],
                            ssem[p]).wait()

    for p in range(S):
      i_start(p, 0, p)

    def loop_body(j, carry):
      for f in range(2):
        cbase = (2 * j + f) * S
        for p in range(S):
          if f == 0:
            @pl.when(j > 0)
            def _():
              s_wait(f, p)
              if with_count:
                c_wait(f, p)
          else:
            s_wait(f, p)
            if with_count:
              c_wait(f, p)
          i_start(jnp.minimum(cbase + S + p, CH - 1), 1 - f, p)
          i_wait(f, p)
          g_start(cbase + p, p)
        for p in range(S):
          g_wait(p)
          s_start(f, p)
          if with_count:
            c_start(f, p)
      return carry

    lax.fori_loop(0, NRND // 2, loop_body, 0)
    for p in range(S):
      s_wait(1, p)
      if with_count:
        c_wait(1, p)
      i_wait(0, p)
    plsc.subcore_barrier()
    pltpu.sync_copy(acc_sh.at[pl.ds(base, RPS)],
                    sum_hbm.at[cid, pl.ds(base, RPS)])
    if with_count:
      pltpu.sync_copy(cnt_sh.at[pl.ds(base, RPS)], cntv)
      @pl.when(cid == 0)
      def _():
        pltpu.sync_copy(cntv, cnt0_hbm.at[pl.ds(base, RPS)])
      @pl.when(cid == 1)
      def _():
        pltpu.sync_copy(cntv, cnt1_hbm.at[pl.ds(base, RPS)])

  return pl.kernel(body, out_type=out_type, mesh=mesh, scratch_types=scratch)


_sc_agg_count = _make_sc_aggregate(True)
_sc_agg = _make_sc_aggregate(False)


def _tc_split_body(ei_ref, src_ref, dst_ref):
  src_ref[...] = ei_ref[0]
  dst_ref[...] = ei_ref[1]


_tc_split = pl.pallas_call(
    _tc_split_body,
    out_shape=[jax.ShapeDtypeStruct((E,), jnp.int32)] * 2,
)


def _tc_layer_body(has_final, sp_ref, cnt0_ref, cnt1_ref, x_ref, Wl_ref,
                   bl_ref, Wr_ref, *rest):
  if has_final:
    Wf_ref, bf_ref, out_ref = rest
  else:
    (out_ref,) = rest
  s = sp_ref[0] + sp_ref[1]
  c = jnp.maximum(cnt0_ref[0, 0] + cnt1_ref[0, 0], 1.0)
  agg = s / c[:, None]
  dn = (((1,), (1,)), ((), ()))
  h = lax.dot_general(agg, Wl_ref[...], dn, preferred_element_type=jnp.float32)
  h = h + bl_ref[...] + lax.dot_general(
      x_ref[...], Wr_ref[...], dn, preferred_element_type=jnp.float32)
  h = jnp.maximum(h, 0.0)
  if has_final:
    h = lax.dot_general(h, Wf_ref[...], dn,
                        preferred_element_type=jnp.float32) + bf_ref[...]
  out_ref[...] = h


def _make_tc_layer(has_final):
  wspec = pl.BlockSpec((D, D), lambda i: (0, 0))
  bspec = pl.BlockSpec((1, D), lambda i: (0, 0))
  cspec = pl.BlockSpec((1, 1, RB), lambda i: (i, 0, 0))
  in_specs = [
      pl.BlockSpec((NC, RB, D), lambda i: (0, i, 0)),
      cspec, cspec,
      pl.BlockSpec((RB, D), lambda i: (i, 0)),
      wspec, bspec, wspec,
  ]
  if has_final:
    in_specs += [wspec, bspec]
  return pl.pallas_call(
      functools.partial(_tc_layer_body, has_final),
      grid=(GRID,),
      in_specs=in_specs,
      out_specs=pl.BlockSpec((RB, D), lambda i: (i, 0)),
      out_shape=jax.ShapeDtypeStruct((N, D), jnp.float32),
  )


_tc_layer = _make_tc_layer(False)
_tc_layer_final = _make_tc_layer(True)


def kernel(x, edge_index, W1l, b1l, W1r, W2l, b2l, W2r, Wf, bf):
  src, dst = _tc_split(edge_index)
  sum1, cnt0, cnt1 = _sc_agg_count(x, src, dst)
  cnt0 = cnt0.reshape(GRID, 1, RB)
  cnt1 = cnt1.reshape(GRID, 1, RB)
  h1 = _tc_layer(sum1, cnt0, cnt1, x, W1l, b1l.reshape(1, D), W1r)
  (sum2,) = _sc_agg(h1, src, dst)
  return _tc_layer_final(sum2, cnt0, cnt1, h1, W2l, b2l.reshape(1, D), W2r,
                         Wf, bf.reshape(1, D))


# speedup vs baseline: 1.0115x; 1.0030x over previous
import functools

import jax
import jax.numpy as jnp
from jax import lax
from jax.experimental import pallas as pl
from jax.experimental.pallas import tpu as pltpu
from jax.experimental.pallas import tpu_sc as plsc

N = 10000
E = 320000
D = 128

NC, NS = 2, 16
NW = NC * NS
EPW = E // NW
K = 40
CH = EPW // K
RPS = 640
RB = 2000
GRID = N // RB


S = 5
NRND = CH // S


def _make_sc_aggregate(with_count):
  mesh = plsc.VectorSubcoreMesh(core_axis_name="c", subcore_axis_name="s")
  out_type = [jax.ShapeDtypeStruct((NC, N, D), jnp.float32)]
  scratch = (
      [pltpu.VMEM((EPW,), jnp.int32)] +
      [pltpu.VMEM((K,), jnp.int32)] * (2 * S) +
      [pltpu.VMEM((K, D), jnp.float32)] * S +
      [pltpu.SemaphoreType.DMA] * (4 * S + 2) +
      [pltpu.VMEM_SHARED((N, D), jnp.float32)]
  )
  if with_count:
    out_type.append(jax.ShapeDtypeStruct((N,), jnp.float32))
    out_type.append(jax.ShapeDtypeStruct((N,), jnp.float32))
    scratch += [
        pltpu.VMEM((48,), jnp.float32),
        pltpu.VMEM((RPS,), jnp.float32),
        pltpu.VMEM_SHARED((N,), jnp.float32),
    ]

  def body(x_hbm, src_hbm, dst_hbm, *rest):
    if with_count:
      sum_hbm, cnt0_hbm, cnt1_hbm = rest[:3]
      rest = rest[3:]
    else:
      (sum_hbm,) = rest[:1]
      rest = rest[1:]
    srcv = rest[0]
    idxd = (rest[1:1 + S], rest[1 + S:1 + 2 * S])
    rows = rest[1 + 2 * S:1 + 3 * S]
    rest = rest[1 + 3 * S:]
    isem = (rest[:S], rest[S:2 * S])
    gsem = rest[2 * S:3 * S]
    ssem = rest[3 * S:4 * S]
    zsem = rest[4 * S]
    xsem = rest[4 * S + 1]
    acc_sh = rest[4 * S + 2]
    if with_count:
      ones, cntv, cnt_sh = rest[4 * S + 3:]
    cid = lax.axis_index("c")
    sid = lax.axis_index("s")
    wid = sid * NC + cid
    base = jnp.minimum(sid * RPS, N - RPS)
    ebase = wid * EPW
    pltpu.async_copy(src_hbm.at[pl.ds(ebase, EPW)], srcv, xsem)
    for p in range(S):
      pltpu.async_copy(dst_hbm.at[pl.ds(ebase + p * K, K)], idxd[0][p],
                       isem[0][p])
    for r in range(K):
      for j in range(D // 16):
        rows[0][r, pl.ds(j * 16, 16)] = jnp.zeros((16,), jnp.float32)
    for t in range(RPS // K):
      pltpu.async_copy(rows[0], acc_sh.at[pl.ds(base + t * K, K)], zsem)
    if with_count:
      for i in range(RPS // 16):
        cntv[pl.ds(i * 16, 16)] = jnp.zeros((16,), jnp.float32)
      pltpu.sync_copy(cntv, cnt_sh.at[pl.ds(base, RPS)])
      for i in range(3):
        ones[pl.ds(i * 16, 16)] = jnp.ones((16,), jnp.float32)
    for t in range(RPS // K):
      pltpu.make_async_copy(rows[0], acc_sh.at[pl.ds(base, K)], zsem).wait()
    pltpu.make_async_copy(src_hbm.at[pl.ds(ebase, EPW)], srcv, xsem).wait()
    plsc.subcore_barrier()

    def i_start(c, f, p):
      pltpu.async_copy(dst_hbm.at[pl.ds(ebase + c * K, K)],
                       idxd[f][p], isem[f][p])

    def i_wait(f, p):
      pltpu.make_async_copy(dst_hbm.at[pl.ds(ebase, K)],
                            idxd[f][p], isem[f][p]).wait()

    def g_start(c, p):
      pltpu.async_copy(x_hbm.at[srcv.at[pl.ds(c * K, K)]], rows[p], gsem[p])

    def g_wait(p):
      pltpu.make_async_copy(x_hbm.at[srcv.at[pl.ds(0, K)]], rows[p],
                            gsem[p]).wait()

    def s_start(f, p):
      pltpu.async_copy(rows[p], acc_sh.at[idxd[f][p]], ssem[p], add=True)

    def s_wait(f, p):
      pltpu.make_async_copy(rows[p], acc_sh.at[idxd[f][p]], ssem[p]).wait()

    def c_start(f, p):
      pltpu.async_copy(ones.at[pl.ds(0, K)], cnt_sh.at[idxd[f][p]],
                       ssem[p], add=True)

    def c_wait(f, p):
      pltpu.make_async_copy(ones.at[pl.ds(0, K)], cnt_sh.at[idxd[f][p]],
                            ssem[p]).wait()

    def loop_body(j, carry):
      for f in range(2):
        cbase = (2 * j + f) * S
        for p in range(S):
          if f == 0:
            @pl.when(j > 0)
            def _():
              s_wait(f, p)
              if with_count:
                c_wait(f, p)
          else:
            s_wait(f, p)
            if with_count:
              c_wait(f, p)
          i_start(jnp.minimum(cbase + S + p, CH - 1), 1 - f, p)
          i_wait(f, p)
          g_start(cbase + p, p)
        for p in range(S):
          g_wait(p)
          s_start(f, p)
          if with_count:
            c_start(f, p)
      return carry

    lax.fori_loop(0, NRND // 2, loop_body, 0)
    for p in range(S):
      s_wait(1, p)
      if with_count:
        c_wait(1, p)
      i_wait(0, p)
    plsc.subcore_barrier()
    pltpu.sync_copy(acc_sh.at[pl.ds(base, RPS)],
                    sum_hbm.at[cid, pl.ds(base, RPS)])
    if with_count:
      pltpu.sync_copy(cnt_sh.at[pl.ds(base, RPS)], cntv)
      @pl.when(cid == 0)
      def _():
        pltpu.sync_copy(cntv, cnt0_hbm.at[pl.ds(base, RPS)])
      @pl.when(cid == 1)
      def _():
        pltpu.sync_copy(cntv, cnt1_hbm.at[pl.ds(base, RPS)])

  return pl.kernel(body, out_type=out_type, mesh=mesh, scratch_types=scratch)


_sc_agg_count = _make_sc_aggregate(True)
_sc_agg = _make_sc_aggregate(False)


def _tc_split_body(ei_ref, src_ref, dst_ref):
  src_ref[...] = ei_ref[0]
  dst_ref[...] = ei_ref[1]


_tc_split = pl.pallas_call(
    _tc_split_body,
    out_shape=[jax.ShapeDtypeStruct((E,), jnp.int32)] * 2,
)


def _tc_layer_body(has_final, sp_ref, cnt0_ref, cnt1_ref, x_ref, Wl_ref,
                   bl_ref, Wr_ref, *rest):
  if has_final:
    Wf_ref, bf_ref, out_ref = rest
  else:
    (out_ref,) = rest
  s = sp_ref[0] + sp_ref[1]
  c = jnp.maximum(cnt0_ref[0, 0] + cnt1_ref[0, 0], 1.0)
  agg = s / c[:, None]
  dn = (((1,), (1,)), ((), ()))
  h = lax.dot_general(agg, Wl_ref[...], dn, preferred_element_type=jnp.float32)
  h = h + bl_ref[...] + lax.dot_general(
      x_ref[...], Wr_ref[...], dn, preferred_element_type=jnp.float32)
  h = jnp.maximum(h, 0.0)
  if has_final:
    h = lax.dot_general(h, Wf_ref[...], dn,
                        preferred_element_type=jnp.float32) + bf_ref[...]
  out_ref[...] = h


def _make_tc_layer(has_final):
  wspec = pl.BlockSpec((D, D), lambda i: (0, 0))
  bspec = pl.BlockSpec((1, D), lambda i: (0, 0))
  cspec = pl.BlockSpec((1, 1, RB), lambda i: (i, 0, 0))
  in_specs = [
      pl.BlockSpec((NC, RB, D), lambda i: (0, i, 0)),
      cspec, cspec,
      pl.BlockSpec((RB, D), lambda i: (i, 0)),
      wspec, bspec, wspec,
  ]
  if has_final:
    in_specs += [wspec, bspec]
  return pl.pallas_call(
      functools.partial(_tc_layer_body, has_final),
      grid=(GRID,),
      in_specs=in_specs,
      out_specs=pl.BlockSpec((RB, D), lambda i: (i, 0)),
      out_shape=jax.ShapeDtypeStruct((N, D), jnp.float32),
  )


_tc_layer = _make_tc_layer(False)
_tc_layer_final = _make_tc_layer(True)


def kernel(x, edge_index, W1l, b1l, W1r, W2l, b2l, W2r, Wf, bf):
  src, dst = _tc_split(edge_index)
  sum1, cnt0, cnt1 = _sc_agg_count(x, src, dst)
  cnt0 = cnt0.reshape(GRID, 1, RB)
  cnt1 = cnt1.reshape(GRID, 1, RB)
  h1 = _tc_layer(sum1, cnt0, cnt1, x, W1l, b1l.reshape(1, D), W1r)
  (sum2,) = _sc_agg(h1, src, dst)
  return _tc_layer_final(sum2, cnt0, cnt1, h1, W2l, b2l.reshape(1, D), W2r,
                         Wf, bf.reshape(1, D))
